# baseline (speedup 1.0000x reference)
import functools

import jax
import jax.numpy as jnp
from jax import lax
from jax.experimental import pallas as pl
from jax.experimental.pallas import tpu as pltpu
from jax.experimental.pallas import tpu_sc as plsc

_SCALE = 1.0 / 32
_PH = 7
_PW = 7
_H = 32
_W = 32
_C = 128
_BAND_ROWS = 18
_LANES = 16
_OUT_SZ = _C * _PH * _PW


def _worker_id():
    return lax.axis_index("c") * 16 + lax.axis_index("s")


def _roi_align_sc(feat_hbm, rois_hbm, out_hbm, fbuf, rois_v, obuf, osem):
    n_roi = out_hbm.shape[0]
    wid = _worker_id()

    my_b = wid & 1
    t = wid >> 1
    my_band = jnp.where(t < 4, 0, jnp.where(t < 8, 1, 2))
    my_slot = jnp.where(t < 4, t, jnp.where(t < 8, t - 4, t - 8))
    nslots_m1 = jnp.where(t < 8, 3, 7)
    band_start = my_band * 7

    pltpu.sync_copy(rois_hbm, rois_v)
    pltpu.sync_copy(feat_hbm.at[my_b, pl.ds(band_start, _BAND_ROWS)], fbuf)

    lane = lax.iota(jnp.int32, _LANES)
    lane49 = lane * 49
    lanef = lane.astype(jnp.float32)
    gf = (lane >> 1).astype(jnp.float32)
    mf = (lane & 1).astype(jnp.float32)

    def compute_roi(n, rv, mycnt):
        cur = mycnt & 1
        obase = pl.multiple_of(cur * _OUT_SZ, _LANES)

        @pl.when(mycnt >= 2)
        def _():
            pltpu.make_async_copy(
                obuf.at[pl.ds(obase, _OUT_SZ)], out_hbm.at[n],
                osem.at[cur]).wait()

        x1s = rv[1]
        y1s = rv[2]
        x2s = rv[3]
        y2s = rv[4]
        bx1 = jnp.minimum(jnp.maximum(x1s * _SCALE, 0.0), _W - 1.0)
        by1 = jnp.minimum(jnp.maximum(y1s * _SCALE, 0.0), _H - 1.0)
        bx2 = jnp.minimum(jnp.maximum(x2s * _SCALE, 0.0), _W - 1.0)
        by2 = jnp.minimum(jnp.maximum(y2s * _SCALE, 0.0), _H - 1.0)

        bwv = jnp.broadcast_to(bx2 - bx1, (_LANES,)) / float(_PW)
        bhv = jnp.broadcast_to(by2 - by1, (_LANES,)) / float(_PH)
        swv = bwv / 2.0
        shv = bhv / 2.0

        syv = jnp.clip(by1 + gf * bhv, 0.0, _H - 1.0)
        yg = syv + shv * 0.5 + mf * shv
        y0v = jnp.clip(yg.astype(jnp.int32), 0, _H - 2)
        wyv = yg - y0v.astype(jnp.float32)
        sxv = jnp.clip(bx1 + gf * bwv, 0.0, _W - 1.0)
        xg = sxv + swv * 0.5 + mf * swv
        x0v = jnp.clip(xg.astype(jnp.int32), 0, _W - 2)
        wxv = xg - x0v.astype(jnp.float32)

        syb = jnp.clip(by1 + lanef * bhv, 0.0, _H - 1.0)
        eyb = jnp.clip(by1 + (lanef + 1.0) * bhv, 0.0, _H - 1.0)
        vyv = jnp.where(syb < eyb, 1.0, 0.0).astype(jnp.float32)
        sxb = jnp.clip(bx1 + lanef * bwv, 0.0, _W - 1.0)
        exb = jnp.clip(bx1 + (lanef + 1.0) * bwv, 0.0, _W - 1.0)
        vxv = jnp.where(sxb < exb, 1.0, 0.0).astype(jnp.float32)

        ylv = y0v - band_start
        yl_s = [ylv[k] for k in range(14)]
        wy_s = [wyv[k] for k in range(14)]
        xl_s = [x0v[k] for k in range(14)]
        wx_s = [wxv[k] for k in range(14)]
        vy_s = [vyv[k] for k in range(7)]
        vx_s = [vxv[k] for k in range(7)]

        for i in range(_PH):
            ya = jnp.clip(yl_s[2 * i], 0, _BAND_ROWS - 2)
            yb = jnp.clip(yl_s[2 * i + 1], 0, _BAND_ROWS - 2)
            wyav = jnp.broadcast_to(wy_s[2 * i], (_LANES,))
            wybv = jnp.broadcast_to(wy_s[2 * i + 1], (_LANES,))
            wya1v = 1.0 - wyav
            wyb1v = 1.0 - wybv
            for j in range(_PW):
                xa = jnp.clip(xl_s[2 * j], 0, _W - 2)
                xb = jnp.clip(xl_s[2 * j + 1], 0, _W - 2)
                wxav = jnp.broadcast_to(wx_s[2 * j], (_LANES,))
                wxbv = jnp.broadcast_to(wx_s[2 * j + 1], (_LANES,))
                wxa1v = 1.0 - wxav
                wxb1v = 1.0 - wxbv
                validv = jnp.broadcast_to(vy_s[i] * vx_s[j], (_LANES,))

                def cbody(c, _, ya=ya, yb=yb, xa=xa, xb=xb, wyav=wyav,
                          wybv=wybv, wya1v=wya1v, wyb1v=wyb1v, wxav=wxav,
                          wxbv=wxbv, wxa1v=wxa1v, wxb1v=wxb1v,
                          validv=validv, i=i, j=j):
                    sl = pl.ds(pl.multiple_of(c * _LANES, _LANES), _LANES)
                    acc = None
                    del sl
                    for (ys, xs, wyv_, wy1v, wxv_, wx1v) in (
                            (ya, xa, wyav, wya1v, wxav, wxa1v),
                            (ya, xb, wyav, wya1v, wxbv, wxb1v),
                            (yb, xa, wybv, wyb1v, wxav, wxa1v),
                            (yb, xb, wybv, wyb1v, wxbv, wxb1v)):
                        sl = pl.ds(pl.multiple_of(c * _LANES, _LANES), _LANES)
                        lt = fbuf[ys, xs, sl]
                        rt = fbuf[ys, xs + 1, sl]
                        lb = fbuf[ys + 1, xs, sl]
                        rb = fbuf[ys + 1, xs + 1, sl]
                        mt = wxv_ * rt + wx1v * lt
                        mb = wxv_ * rb + wx1v * lb
                        pix = wyv_ * mb + wy1v * mt
                        acc = pix if acc is None else jnp.maximum(acc, pix)
                    off = pl.multiple_of(
                        obase + (i * 7 + j) * _C + c * _LANES, _LANES)
                    obuf[pl.ds(off, _LANES)] = acc * validv
                    return 0

                lax.fori_loop(0, 8, cbody, 0, unroll=2)
        pltpu.async_copy(obuf.at[pl.ds(obase, _OUT_SZ)], out_hbm.at[n],
                         osem.at[cur])

    def roi_body(n, carry):
        cnt, mycnt = carry
        rv = rois_v[pl.ds(pl.multiple_of(n * _LANES, _LANES), _LANES)]
        b_i = rv[0].astype(jnp.int32)
        by1c = jnp.minimum(jnp.maximum(rv[2] * _SCALE, 0.0), _H - 1.0)
        band_r = jnp.where(by1c < 7.0, 0, jnp.where(by1c < 14.0, 1, 2))
        mine_class = (b_i == my_b) & (band_r == my_band)
        slot_ok = (cnt & nslots_m1) == my_slot
        claim = mine_class & slot_ok

        @pl.when(claim)
        def _():
            compute_roi(n, rv, mycnt)

        return (cnt + mine_class.astype(jnp.int32),
                mycnt + claim.astype(jnp.int32))

    _, my_total = lax.fori_loop(0, n_roi, roi_body,
                                (jnp.int32(0), jnp.int32(0)))

    @pl.when(my_total >= 1)
    def _():
        lastb = (my_total - 1) & 1
        pltpu.make_async_copy(
            obuf.at[pl.ds(pl.multiple_of(lastb * _OUT_SZ, 8), _OUT_SZ)],
            out_hbm.at[0], osem.at[lastb]).wait()

    @pl.when(my_total >= 2)
    def _():
        prevb = my_total & 1
        pltpu.make_async_copy(
            obuf.at[pl.ds(pl.multiple_of(prevb * _OUT_SZ, 8), _OUT_SZ)],
            out_hbm.at[0], osem.at[prevb]).wait()


@jax.jit
def kernel(features, rois):
    b, c, h, w = features.shape
    n_roi = rois.shape[0]
    feat_nhwc = jnp.transpose(features, (0, 2, 3, 1))
    rois_t = jnp.pad(rois, ((0, 0), (0, _LANES - 5))).reshape(-1)

    mesh = plsc.VectorSubcoreMesh(core_axis_name="c", subcore_axis_name="s",
                                  num_cores=2, num_subcores=16)
    run = pl.kernel(
        _roi_align_sc,
        out_type=jax.ShapeDtypeStruct((n_roi, c * _PH * _PW), jnp.float32),
        mesh=mesh,
        compiler_params=pltpu.CompilerParams(needs_layout_passes=False),
        scratch_types=[
            pltpu.VMEM((_BAND_ROWS, w, c), jnp.float32),
            pltpu.VMEM((n_roi * _LANES,), jnp.float32),
            pltpu.VMEM((2 * _OUT_SZ,), jnp.float32),
            pltpu.SemaphoreType.DMA((2,)),
        ],
    )
    out = run(feat_nhwc, rois_t).reshape(n_roi, _PH, _PW, c)
    return jnp.transpose(out, (0, 3, 1, 2))


# speedup vs baseline: 1.4630x; 1.4630x over previous
import functools

import jax
import jax.numpy as jnp
from jax import lax
from jax.experimental import pallas as pl
from jax.experimental.pallas import tpu as pltpu
from jax.experimental.pallas import tpu_sc as plsc

_SCALE = 1.0 / 32
_PH = 7
_PW = 7
_H = 32
_W = 32
_C = 128
_BAND_ROWS = 18
_LANES = 16
_OUT_SZ = _C * _PH * _PW


def _worker_id():
    return lax.axis_index("c") * 16 + lax.axis_index("s")


def _roi_align_sc(feat_hbm, rois_hbm, out_hbm, fbuf, rois_v, obuf, osem):
    n_roi = out_hbm.shape[0]
    wid = _worker_id()

    my_b = wid & 1
    t = wid >> 1
    my_band = jnp.where(t < 4, 0, jnp.where(t < 8, 1, 2))
    my_slot = jnp.where(t < 4, t, jnp.where(t < 8, t - 4, t - 8))
    nslots_m1 = jnp.where(t < 8, 3, 7)
    band_start = my_band * 7

    pltpu.sync_copy(rois_hbm, rois_v)
    pltpu.sync_copy(feat_hbm.at[my_b, pl.ds(band_start, _BAND_ROWS)], fbuf)

    lane = lax.iota(jnp.int32, _LANES)
    lane49 = lane * 49
    lanef = lane.astype(jnp.float32)
    gf = (lane >> 1).astype(jnp.float32)
    mf = (lane & 1).astype(jnp.float32)

    def compute_roi(n, rv, mycnt):
        cur = mycnt & 1
        obase = pl.multiple_of(cur * _OUT_SZ, _LANES)

        @pl.when(mycnt >= 2)
        def _():
            pltpu.make_async_copy(
                obuf.at[pl.ds(obase, _OUT_SZ)], out_hbm.at[n],
                osem.at[cur]).wait()

        x1s = rv[1]
        y1s = rv[2]
        x2s = rv[3]
        y2s = rv[4]
        bx1 = jnp.minimum(jnp.maximum(x1s * _SCALE, 0.0), _W - 1.0)
        by1 = jnp.minimum(jnp.maximum(y1s * _SCALE, 0.0), _H - 1.0)
        bx2 = jnp.minimum(jnp.maximum(x2s * _SCALE, 0.0), _W - 1.0)
        by2 = jnp.minimum(jnp.maximum(y2s * _SCALE, 0.0), _H - 1.0)

        bwv = jnp.broadcast_to(bx2 - bx1, (_LANES,)) / float(_PW)
        bhv = jnp.broadcast_to(by2 - by1, (_LANES,)) / float(_PH)
        swv = bwv / 2.0
        shv = bhv / 2.0

        syv = jnp.clip(by1 + gf * bhv, 0.0, _H - 1.0)
        yg = syv + shv * 0.5 + mf * shv
        y0v = jnp.clip(yg.astype(jnp.int32), 0, _H - 2)
        wyv = yg - y0v.astype(jnp.float32)
        sxv = jnp.clip(bx1 + gf * bwv, 0.0, _W - 1.0)
        xg = sxv + swv * 0.5 + mf * swv
        x0v = jnp.clip(xg.astype(jnp.int32), 0, _W - 2)
        wxv = xg - x0v.astype(jnp.float32)

        syb = jnp.clip(by1 + lanef * bhv, 0.0, _H - 1.0)
        eyb = jnp.clip(by1 + (lanef + 1.0) * bhv, 0.0, _H - 1.0)
        vyv = jnp.where(syb < eyb, 1.0, 0.0).astype(jnp.float32)
        sxb = jnp.clip(bx1 + lanef * bwv, 0.0, _W - 1.0)
        exb = jnp.clip(bx1 + (lanef + 1.0) * bwv, 0.0, _W - 1.0)
        vxv = jnp.where(sxb < exb, 1.0, 0.0).astype(jnp.float32)

        ylv = y0v - band_start
        yl_s = [ylv[k] for k in range(14)]
        wy_s = [wyv[k] for k in range(14)]
        xl_s = [x0v[k] for k in range(14)]
        wx_s = [wxv[k] for k in range(14)]
        vy_s = [vyv[k] for k in range(7)]
        vx_s = [vxv[k] for k in range(7)]

        for i in range(_PH):
            ya = jnp.clip(yl_s[2 * i], 0, _BAND_ROWS - 2)
            yb = jnp.clip(yl_s[2 * i + 1], 0, _BAND_ROWS - 2)
            wyav = jnp.broadcast_to(wy_s[2 * i], (_LANES,))
            wybv = jnp.broadcast_to(wy_s[2 * i + 1], (_LANES,))
            wya1v = 1.0 - wyav
            wyb1v = 1.0 - wybv
            for j in range(_PW):
                xa = jnp.clip(xl_s[2 * j], 0, _W - 2)
                xb = jnp.clip(xl_s[2 * j + 1], 0, _W - 2)
                wxav = jnp.broadcast_to(wx_s[2 * j], (_LANES,))
                wxbv = jnp.broadcast_to(wx_s[2 * j + 1], (_LANES,))
                wxa1v = 1.0 - wxav
                wxb1v = 1.0 - wxbv
                validv = jnp.broadcast_to(vy_s[i] * vx_s[j], (_LANES,))

                def cbody(c, _, ya=ya, yb=yb, xa=xa, xb=xb, wyav=wyav,
                          wybv=wybv, wya1v=wya1v, wyb1v=wyb1v, wxav=wxav,
                          wxbv=wxbv, wxa1v=wxa1v, wxb1v=wxb1v,
                          validv=validv, i=i, j=j):
                    sl = pl.ds(pl.multiple_of(c * _LANES, _LANES), _LANES)
                    acc = None
                    del sl
                    for (ys, xs, wyv_, wy1v, wxv_, wx1v) in (
                            (ya, xa, wyav, wya1v, wxav, wxa1v),
                            (ya, xb, wyav, wya1v, wxbv, wxb1v),
                            (yb, xa, wybv, wyb1v, wxav, wxa1v),
                            (yb, xb, wybv, wyb1v, wxbv, wxb1v)):
                        sl = pl.ds(pl.multiple_of(c * _LANES, _LANES), _LANES)
                        lt = fbuf[ys, xs, sl]
                        rt = fbuf[ys, xs + 1, sl]
                        lb = fbuf[ys + 1, xs, sl]
                        rb = fbuf[ys + 1, xs + 1, sl]
                        mt = wxv_ * rt + wx1v * lt
                        mb = wxv_ * rb + wx1v * lb
                        pix = wyv_ * mb + wy1v * mt
                        acc = pix if acc is None else jnp.maximum(acc, pix)
                    off = pl.multiple_of(
                        obase + (i * 7 + j) * _C + c * _LANES, _LANES)
                    obuf[pl.ds(off, _LANES)] = acc * validv
                    return 0

                lax.fori_loop(0, 8, cbody, 0)
        pltpu.async_copy(obuf.at[pl.ds(obase, _OUT_SZ)], out_hbm.at[n],
                         osem.at[cur])

    def roi_body(n, carry):
        cnt, mycnt = carry
        rv = rois_v[pl.ds(pl.multiple_of(n * _LANES, _LANES), _LANES)]
        b_i = rv[0].astype(jnp.int32)
        by1c = jnp.minimum(jnp.maximum(rv[2] * _SCALE, 0.0), _H - 1.0)
        band_r = jnp.where(by1c < 7.0, 0, jnp.where(by1c < 14.0, 1, 2))
        mine_class = (b_i == my_b) & (band_r == my_band)
        slot_ok = (cnt & nslots_m1) == my_slot
        claim = mine_class & slot_ok

        @pl.when(claim)
        def _():
            compute_roi(n, rv, mycnt)

        return (cnt + mine_class.astype(jnp.int32),
                mycnt + claim.astype(jnp.int32))

    _, my_total = lax.fori_loop(0, n_roi, roi_body,
                                (jnp.int32(0), jnp.int32(0)))

    @pl.when(my_total >= 1)
    def _():
        lastb = (my_total - 1) & 1
        pltpu.make_async_copy(
            obuf.at[pl.ds(pl.multiple_of(lastb * _OUT_SZ, 8), _OUT_SZ)],
            out_hbm.at[0], osem.at[lastb]).wait()

    @pl.when(my_total >= 2)
    def _():
        prevb = my_total & 1
        pltpu.make_async_copy(
            obuf.at[pl.ds(pl.multiple_of(prevb * _OUT_SZ, 8), _OUT_SZ)],
            out_hbm.at[0], osem.at[prevb]).wait()


@jax.jit
def kernel(features, rois):
    b, c, h, w = features.shape
    n_roi = rois.shape[0]
    feat_nhwc = jnp.transpose(features, (0, 2, 3, 1))
    rois_t = jnp.pad(rois, ((0, 0), (0, _LANES - 5))).reshape(-1)

    mesh = plsc.VectorSubcoreMesh(core_axis_name="c", subcore_axis_name="s",
                                  num_cores=2, num_subcores=16)
    run = pl.kernel(
        _roi_align_sc,
        out_type=jax.ShapeDtypeStruct((n_roi, c * _PH * _PW), jnp.float32),
        mesh=mesh,
        compiler_params=pltpu.CompilerParams(needs_layout_passes=False),
        scratch_types=[
            pltpu.VMEM((_BAND_ROWS, w, c), jnp.float32),
            pltpu.VMEM((n_roi * _LANES,), jnp.float32),
            pltpu.VMEM((2 * _OUT_SZ,), jnp.float32),
            pltpu.SemaphoreType.DMA((2,)),
        ],
    )
    out = run(feat_nhwc, rois_t).reshape(n_roi, _PH, _PW, c)
    return jnp.transpose(out, (0, 3, 1, 2))


# speedup vs baseline: 1.7478x; 1.1946x over previous
import functools

import jax
import jax.numpy as jnp
from jax import lax
from jax.experimental import pallas as pl
from jax.experimental.pallas import tpu as pltpu
from jax.experimental.pallas import tpu_sc as plsc

_SCALE = 1.0 / 32
_PH = 7
_PW = 7
_H = 32
_W = 32
_C = 128
_BAND_ROWS = 18
_LANES = 16
_OUT_SZ = _C * _PH * _PW


def _worker_id():
    return lax.axis_index("c") * 16 + lax.axis_index("s")


def _roi_align_sc(feat_hbm, rois_hbm, out_hbm, fbuf, rois_v, obuf, osem):
    n_roi = out_hbm.shape[0]
    wid = _worker_id()

    my_b = wid & 1
    t = wid >> 1
    my_band = jnp.where(t < 4, 0, jnp.where(t < 8, 1, 2))
    my_slot = jnp.where(t < 4, t, jnp.where(t < 8, t - 4, t - 8))
    nslots_m1 = jnp.where(t < 8, 3, 7)
    band_start = my_band * 7

    pltpu.sync_copy(rois_hbm, rois_v)
    pltpu.sync_copy(feat_hbm.at[my_b, pl.ds(band_start, _BAND_ROWS)], fbuf)

    lane = lax.iota(jnp.int32, _LANES)
    lane49 = lane * 49
    lanef = lane.astype(jnp.float32)
    gf = (lane >> 1).astype(jnp.float32)
    mf = (lane & 1).astype(jnp.float32)

    def compute_roi(n, rv, mycnt):
        cur = mycnt & 1
        obase = pl.multiple_of(cur * _OUT_SZ, _LANES)

        @pl.when(mycnt >= 2)
        def _():
            pltpu.make_async_copy(
                obuf.at[pl.ds(obase, _OUT_SZ)], out_hbm.at[n],
                osem.at[cur]).wait()

        x1s = rv[1]
        y1s = rv[2]
        x2s = rv[3]
        y2s = rv[4]
        bx1 = jnp.minimum(jnp.maximum(x1s * _SCALE, 0.0), _W - 1.0)
        by1 = jnp.minimum(jnp.maximum(y1s * _SCALE, 0.0), _H - 1.0)
        bx2 = jnp.minimum(jnp.maximum(x2s * _SCALE, 0.0), _W - 1.0)
        by2 = jnp.minimum(jnp.maximum(y2s * _SCALE, 0.0), _H - 1.0)

        bwv = jnp.broadcast_to(bx2 - bx1, (_LANES,)) / float(_PW)
        bhv = jnp.broadcast_to(by2 - by1, (_LANES,)) / float(_PH)
        swv = bwv / 2.0
        shv = bhv / 2.0

        syv = jnp.clip(by1 + gf * bhv, 0.0, _H - 1.0)
        yg = syv + shv * 0.5 + mf * shv
        y0v = jnp.clip(yg.astype(jnp.int32), 0, _H - 2)
        wyv = yg - y0v.astype(jnp.float32)
        sxv = jnp.clip(bx1 + gf * bwv, 0.0, _W - 1.0)
        xg = sxv + swv * 0.5 + mf * swv
        x0v = jnp.clip(xg.astype(jnp.int32), 0, _W - 2)
        wxv = xg - x0v.astype(jnp.float32)

        syb = jnp.clip(by1 + lanef * bhv, 0.0, _H - 1.0)
        eyb = jnp.clip(by1 + (lanef + 1.0) * bhv, 0.0, _H - 1.0)
        vyv = jnp.where(syb < eyb, 1.0, 0.0).astype(jnp.float32)
        sxb = jnp.clip(bx1 + lanef * bwv, 0.0, _W - 1.0)
        exb = jnp.clip(bx1 + (lanef + 1.0) * bwv, 0.0, _W - 1.0)
        vxv = jnp.where(sxb < exb, 1.0, 0.0).astype(jnp.float32)

        ylv = y0v - band_start
        yl_s = [ylv[k] for k in range(14)]
        wy_s = [wyv[k] for k in range(14)]
        xl_s = [x0v[k] for k in range(14)]
        wx_s = [wxv[k] for k in range(14)]
        vy_s = [vyv[k] for k in range(7)]
        vx_s = [vxv[k] for k in range(7)]

        for i in range(_PH):
            ya = jnp.clip(yl_s[2 * i], 0, _BAND_ROWS - 2)
            yb = jnp.clip(yl_s[2 * i + 1], 0, _BAND_ROWS - 2)
            wyav = jnp.broadcast_to(wy_s[2 * i], (_LANES,))
            wybv = jnp.broadcast_to(wy_s[2 * i + 1], (_LANES,))
            wya1v = 1.0 - wyav
            wyb1v = 1.0 - wybv
            for j in range(_PW):
                xa = jnp.clip(xl_s[2 * j], 0, _W - 2)
                xb = jnp.clip(xl_s[2 * j + 1], 0, _W - 2)
                wxav = jnp.broadcast_to(wx_s[2 * j], (_LANES,))
                wxbv = jnp.broadcast_to(wx_s[2 * j + 1], (_LANES,))
                wxa1v = 1.0 - wxav
                wxb1v = 1.0 - wxbv
                validv = jnp.broadcast_to(vy_s[i] * vx_s[j], (_LANES,))

                def cbody(c, _, ya=ya, yb=yb, xa=xa, xb=xb, wyav=wyav,
                          wybv=wybv, wya1v=wya1v, wyb1v=wyb1v, wxav=wxav,
                          wxbv=wxbv, wxa1v=wxa1v, wxb1v=wxb1v,
                          validv=validv, i=i, j=j):
                    sl = pl.ds(pl.multiple_of(c * _LANES, _LANES), _LANES)
                    acc = None
                    del sl
                    for (ys, xs, wyv_, wy1v, wxv_, wx1v) in (
                            (ya, xa, wyav, wya1v, wxav, wxa1v),
                            (ya, xb, wyav, wya1v, wxbv, wxb1v),
                            (yb, xa, wybv, wyb1v, wxav, wxa1v),
                            (yb, xb, wybv, wyb1v, wxbv, wxb1v)):
                        sl = pl.ds(pl.multiple_of(c * _LANES, _LANES), _LANES)
                        lt = fbuf[ys, xs, sl]
                        rt = fbuf[ys, xs + 1, sl]
                        lb = fbuf[ys + 1, xs, sl]
                        rb = fbuf[ys + 1, xs + 1, sl]
                        mt = wxv_ * rt + wx1v * lt
                        mb = wxv_ * rb + wx1v * lb
                        pix = wyv_ * mb + wy1v * mt
                        acc = pix if acc is None else jnp.maximum(acc, pix)
                    off = pl.multiple_of(
                        obase + (i * 7 + j) * _C + c * _LANES, _LANES)
                    obuf[pl.ds(off, _LANES)] = acc * validv
                    return 0

                plsc.parallel_loop(0, 8)(
                    functools.partial(cbody, _=0))
        pltpu.async_copy(obuf.at[pl.ds(obase, _OUT_SZ)], out_hbm.at[n],
                         osem.at[cur])

    def roi_body(n, carry):
        cnt, mycnt = carry
        rv = rois_v[pl.ds(pl.multiple_of(n * _LANES, _LANES), _LANES)]
        b_i = rv[0].astype(jnp.int32)
        by1c = jnp.minimum(jnp.maximum(rv[2] * _SCALE, 0.0), _H - 1.0)
        band_r = jnp.where(by1c < 7.0, 0, jnp.where(by1c < 14.0, 1, 2))
        mine_class = (b_i == my_b) & (band_r == my_band)
        slot_ok = (cnt & nslots_m1) == my_slot
        claim = mine_class & slot_ok

        @pl.when(claim)
        def _():
            compute_roi(n, rv, mycnt)

        return (cnt + mine_class.astype(jnp.int32),
                mycnt + claim.astype(jnp.int32))

    _, my_total = lax.fori_loop(0, n_roi, roi_body,
                                (jnp.int32(0), jnp.int32(0)))

    @pl.when(my_total >= 1)
    def _():
        lastb = (my_total - 1) & 1
        pltpu.make_async_copy(
            obuf.at[pl.ds(pl.multiple_of(lastb * _OUT_SZ, 8), _OUT_SZ)],
            out_hbm.at[0], osem.at[lastb]).wait()

    @pl.when(my_total >= 2)
    def _():
        prevb = my_total & 1
        pltpu.make_async_copy(
            obuf.at[pl.ds(pl.multiple_of(prevb * _OUT_SZ, 8), _OUT_SZ)],
            out_hbm.at[0], osem.at[prevb]).wait()


@jax.jit
def kernel(features, rois):
    b, c, h, w = features.shape
    n_roi = rois.shape[0]
    feat_nhwc = jnp.transpose(features, (0, 2, 3, 1))
    rois_t = jnp.pad(rois, ((0, 0), (0, _LANES - 5))).reshape(-1)

    mesh = plsc.VectorSubcoreMesh(core_axis_name="c", subcore_axis_name="s",
                                  num_cores=2, num_subcores=16)
    run = pl.kernel(
        _roi_align_sc,
        out_type=jax.ShapeDtypeStruct((n_roi, c * _PH * _PW), jnp.float32),
        mesh=mesh,
        compiler_params=pltpu.CompilerParams(needs_layout_passes=False),
        scratch_types=[
            pltpu.VMEM((_BAND_ROWS, w, c), jnp.float32),
            pltpu.VMEM((n_roi * _LANES,), jnp.float32),
            pltpu.VMEM((2 * _OUT_SZ,), jnp.float32),
            pltpu.SemaphoreType.DMA((2,)),
        ],
    )
    out = run(feat_nhwc, rois_t).reshape(n_roi, _PH, _PW, c)
    return jnp.transpose(out, (0, 3, 1, 2))
